# Initial kernel scaffold; baseline (speedup 1.0000x reference)
import jax
import jax.numpy as jnp
from jax.experimental import pallas as pl


def kernel(x, pos, edge_attr, W1, W2, W3, W4, W5, W6, W7, gamma1, gamma2, gamma3, gamma4, gamma5, gamma6, gamma7, beta1, beta2, beta3, beta4, beta5, beta6, beta7, fc_w, edge_index, batch):
    raise NotImplementedError("write your pallas kernel here")


# speedup vs baseline: 1.9653x; 1.9653x over previous
import jax
import jax.numpy as jnp
from jax import lax
from jax.experimental import pallas as pl
from jax.experimental.pallas import tpu as pltpu
from jax.experimental.pallas import tpu_sc as plsc

N_NODES = 50000
N_EDGES = 800000
B = 16
NX = 22
NY = 22
VSX = 16.0 / 346.0
VSY = 12.0 / 260.0
C = B * NX * NY

NC, NS = 2, 16
NW = NC * NS
CH = 128
NEG = -3.0e38

_SC_PARAMS = pltpu.CompilerParams(use_tc_tiling_on_sc=False)


def _mesh():
    return plsc.VectorSubcoreMesh(
        core_axis_name="c", subcore_axis_name="s", num_cores=NC, num_subcores=NS)


_USE_SC_GATHER = True
_USE_SC_SCATTER = True
_USE_SC_SEGMAX = True


def _sc_gather(table, idx, D, dtype, use=None):
    if not (_USE_SC_GATHER if use is None else use):
        return table[idx]
    Dp = -(-D // 16) * 16
    if Dp != D:
        table = jnp.pad(table, ((0, 0), (0, Dp - D)))
    out = _sc_gather_real(table, idx, Dp, dtype)
    return out[:, :D] if Dp != D else out


def _sc_gather_real(table, idx, D, dtype):
    M = idx.shape[0]
    per = M // NW
    assert M % NW == 0 and per % 8 == 0
    nfull, tail = divmod(per, CH)
    assert tail % 8 == 0

    def body(table_h, idx_h, out_h, idxA, idxB, rowsA, rowsB, idxT, rowsT,
             semA, semB):
        cid = lax.axis_index("c")
        sid = lax.axis_index("s")
        base = (sid * NC + cid) * per

        def idx_load(buf, c):
            pltpu.sync_copy(idx_h.at[pl.ds(base + c * CH, CH)], buf)

        def g_start(ib, rb, sem):
            pltpu.async_copy(table_h.at[ib], rb, sem)

        def g_wait(ib, rb, sem):
            pltpu.make_async_copy(table_h.at[ib], rb, sem).wait()

        def store(rb, c):
            pltpu.sync_copy(rb, out_h.at[pl.ds(base + c * CH, CH), :])

        del idxB, rowsB, semB, g_start, g_wait

        @pl.loop(0, nfull)
        def _(c):
            idx_load(idxA, c)
            pltpu.async_copy(table_h.at[idxA], rowsA, semA).wait()
            store(rowsA, c)

        if tail:
            pltpu.sync_copy(idx_h.at[pl.ds(base + nfull * CH, tail)], idxT)
            pltpu.async_copy(table_h.at[idxT], rowsT, semA).wait()
            pltpu.sync_copy(rowsT, out_h.at[pl.ds(base + nfull * CH, tail), :])

    fn = pl.kernel(
        body,
        out_type=jax.ShapeDtypeStruct((M, D), dtype),
        mesh=_mesh(),
        compiler_params=_SC_PARAMS,
        scratch_types=[
            pltpu.VMEM((CH,), jnp.int32),
            pltpu.VMEM((CH,), jnp.int32),
            pltpu.VMEM((CH, D), dtype),
            pltpu.VMEM((CH, D), dtype),
            pltpu.VMEM((tail or 8,), jnp.int32),
            pltpu.VMEM((tail or 8, D), dtype),
            pltpu.SemaphoreType.DMA,
            pltpu.SemaphoreType.DMA,
        ],
    )
    return fn(table, idx)


def _sc_scatter_add(msg, dst, nrows):
    if not _USE_SC_SCATTER:
        p0 = jax.ops.segment_sum(msg, dst, num_segments=nrows)
        return jnp.stack([p0, jnp.zeros_like(p0)])
    M, P = msg.shape
    assert P % 16 == 0 and nrows % NS == 0
    per = M // NW
    assert M % NW == 0 and per % 8 == 0
    nfull, tail = divmod(per, CH)
    assert tail % 8 == 0
    rows_t = nrows // NS
    znf, ztail = divmod(rows_t, CH)

    def body(msg_h, dst_h, out_h, acc, idxA, idxB, rowsA, rowsB, idxT, rowsT,
             zb, semA, semB):
        cid = lax.axis_index("c")
        sid = lax.axis_index("s")
        base = cid * (M // NC) + sid * per
        r0 = sid * rows_t

        for r in range(CH):
            for col in range(0, P, 16):
                zb[r, pl.ds(col, 16)] = jnp.zeros((16,), jnp.float32)

        @pl.loop(0, znf)
        def _(j):
            pltpu.sync_copy(zb, acc.at[pl.ds(r0 + j * CH, CH), :])

        if ztail:
            pltpu.sync_copy(zb.at[pl.ds(0, ztail), :],
                            acc.at[pl.ds(r0 + znf * CH, ztail), :])

        plsc.subcore_barrier()

        def idx_load(buf, c):
            pltpu.sync_copy(dst_h.at[pl.ds(base + c * CH, CH)], buf)

        def m_start(rb, c, sem):
            pltpu.async_copy(msg_h.at[pl.ds(base + c * CH, CH), :], rb, sem)

        def m_wait(rb, c, sem):
            pltpu.make_async_copy(
                msg_h.at[pl.ds(base + c * CH, CH), :], rb, sem).wait()

        def scat(ib, rb):
            pltpu.sync_copy(rb, acc.at[ib], add=True)

        if nfull > 0:
            idx_load(idxA, 0)
            m_start(rowsA, 0, semA)

            @pl.loop(0, (nfull + 1) // 2)
            def _(t):
                c0 = 2 * t
                c1 = c0 + 1

                @pl.when(c1 < nfull)
                def _():
                    idx_load(idxB, c1)
                    m_start(rowsB, c1, semB)

                m_wait(rowsA, c0, semA)
                scat(idxA, rowsA)

                @pl.when(c1 < nfull)
                def _():
                    @pl.when(c1 + 1 < nfull)
                    def _():
                        idx_load(idxA, c1 + 1)
                        m_start(rowsA, c1 + 1, semA)

                    m_wait(rowsB, c1, semB)
                    scat(idxB, rowsB)

        if tail:
            pltpu.sync_copy(dst_h.at[pl.ds(base + nfull * CH, tail)], idxT)
            pltpu.async_copy(
                msg_h.at[pl.ds(base + nfull * CH, tail), :], rowsT, semA).wait()
            pltpu.sync_copy(rowsT, acc.at[idxT], add=True)

        plsc.subcore_barrier()

        @pl.loop(0, znf)
        def _(j):
            pltpu.sync_copy(acc.at[pl.ds(r0 + j * CH, CH), :],
                            out_h.at[cid, pl.ds(r0 + j * CH, CH), :])

        if ztail:
            pltpu.sync_copy(acc.at[pl.ds(r0 + znf * CH, ztail), :],
                            out_h.at[cid, pl.ds(r0 + znf * CH, ztail), :])

    fn = pl.kernel(
        body,
        out_type=jax.ShapeDtypeStruct((NC, nrows, P), jnp.float32),
        mesh=_mesh(),
        compiler_params=_SC_PARAMS,
        scratch_types=[
            pltpu.VMEM_SHARED((nrows, P), jnp.float32),
            pltpu.VMEM((CH,), jnp.int32),
            pltpu.VMEM((CH,), jnp.int32),
            pltpu.VMEM((CH, P), jnp.float32),
            pltpu.VMEM((CH, P), jnp.float32),
            pltpu.VMEM((tail or 8,), jnp.int32),
            pltpu.VMEM((tail or 8, P), jnp.float32),
            pltpu.VMEM((CH, P), jnp.float32),
            pltpu.SemaphoreType.DMA,
            pltpu.SemaphoreType.DMA,
        ],
    )
    return fn(msg, dst)


def _sc_segmax(hpad, clpad, tb, cps):
    if not _USE_SC_SEGMAX:
        n = hpad.shape[0] - 512
        p0 = jax.ops.segment_max(hpad[:n], clpad[:n], num_segments=B * cps)
        p0 = jnp.maximum(p0, NEG)
        return jnp.stack([p0, jnp.full_like(p0, NEG)])
    CHN = 512

    def body(h_h, cl_h, tb_h, out_h, acc, hbuf, clbuf, tbv, sem):
        cid = lax.axis_index("c")
        sid = lax.axis_index("s")
        wid = sid * NC + cid
        s = wid // 2
        half = wid % 2
        lo = s * cps

        pltpu.sync_copy(tb_h.at[pl.ds(pl.multiple_of(wid * 16, 8), 16)], tbv)
        bvec = tbv[pl.ds(0, 16)]
        tstart = bvec[0]
        tend = bvec[1]

        @pl.loop(0, cps)
        def _(j):
            acc[j, pl.ds(0, 16)] = jnp.full((16,), NEG, jnp.float32)
            acc[j, pl.ds(16, 16)] = jnp.full((16,), NEG, jnp.float32)

        nch = (tend - tstart + CHN - 1) // CHN

        @pl.loop(0, nch)
        def _(c):
            r0 = pl.multiple_of(tstart + c * CHN, 8)
            pltpu.async_copy(h_h.at[pl.ds(r0, CHN), :], hbuf, sem)
            pltpu.sync_copy(cl_h.at[pl.ds(r0, CHN)], clbuf)
            pltpu.make_async_copy(h_h.at[pl.ds(r0, CHN), :], hbuf, sem).wait()

            @pl.loop(0, CHN // 16)
            def _(v):
                clvec = clbuf[pl.ds(v * 16, 16)]
                for lane in range(16):
                    clv = clvec[lane]
                    i = v * 16 + lane

                    @pl.when((clv >= lo) & (clv < lo + cps))
                    def _():
                        j = clv - lo
                        acc[j, pl.ds(0, 16)] = jnp.maximum(
                            acc[j, pl.ds(0, 16)], hbuf[i, pl.ds(0, 16)])
                        acc[j, pl.ds(16, 16)] = jnp.maximum(
                            acc[j, pl.ds(16, 16)], hbuf[i, pl.ds(16, 16)])

        pltpu.sync_copy(acc, out_h.at[half, pl.ds(s * cps, cps), :])

    fn = pl.kernel(
        body,
        out_type=jax.ShapeDtypeStruct((2, B * cps, 32), jnp.float32),
        mesh=_mesh(),
        compiler_params=_SC_PARAMS,
        scratch_types=[
            pltpu.VMEM((cps, 32), jnp.float32),
            pltpu.VMEM((CHN, 32), jnp.float32),
            pltpu.VMEM((CHN,), jnp.int32),
            pltpu.VMEM((16,), jnp.int32),
            pltpu.SemaphoreType.DMA,
        ],
    )
    return fn(hpad, clpad, tb)


BKE = 3200
BKN = 2000
BKC = 968


def _pad128(row, width):
    return jnp.concatenate(
        [row, jnp.zeros((1, 128 - width), jnp.float32)], axis=1)


def _spline_msg(xs, u0, u1, u2, Wc, a, b):
    if a == 1:
        t = xs * Wc[0:1, :]
    else:
        t = jnp.dot(xs, Wc, preferred_element_type=jnp.float32)
    out = jnp.zeros((xs.shape[0], b), jnp.float32)
    for k in range(8):
        w = jnp.ones((xs.shape[0], 1), jnp.float32)
        w = w * (u0 if k & 1 else 1.0 - u0)
        w = w * (u1 if k & 2 else 1.0 - u1)
        w = w * (u2 if k & 4 else 1.0 - u2)
        out = out + w * t[:, k * b:(k + 1) * b]
    return out


def _tc_edge_fine(xs, ea, Wc, a, b, with_ones):
    P = 16 if b <= 8 or with_ones else b

    def body(xs_ref, ea_ref, wc_ref, out_ref):
        u = jnp.clip(ea_ref[...], 0.0, 1.0)
        msg = _spline_msg(xs_ref[...], u[:, 0:1], u[:, 1:2], u[:, 2:3],
                          wc_ref[...], a, b)
        parts = [msg]
        if with_ones:
            parts.append(jnp.ones((BKE, 1), jnp.float32))
        pad = P - sum(p.shape[1] for p in parts)
        if pad:
            parts.append(jnp.zeros((BKE, pad), jnp.float32))
        out_ref[...] = jnp.concatenate(parts, axis=1)

    return pl.pallas_call(
        body,
        out_shape=jax.ShapeDtypeStruct((N_EDGES, P), jnp.float32),
        grid=(N_EDGES // BKE,),
        in_specs=[pl.BlockSpec((BKE, a), lambda i: (i, 0)),
                  pl.BlockSpec((BKE, 3), lambda i: (i, 0)),
                  pl.BlockSpec((a, 8 * b), lambda i: (0, 0))],
        out_specs=pl.BlockSpec((BKE, P), lambda i: (i, 0)),
    )(xs, ea, Wc)


def _tc_edge_coarse(xs, rel, em, inv2, Wc, with_ones):
    b = 32
    P = 48 if with_ones else 32

    def body(xs_ref, rel_ref, em_ref, inv_ref, wc_ref, out_ref):
        iv = inv_ref[0, 0]
        u = jnp.clip(rel_ref[...] * iv + 0.5, 0.0, 1.0)
        msg = _spline_msg(xs_ref[...], u[:, 0:1], u[:, 1:2], u[:, 2:3],
                          wc_ref[...], 32, b)
        e = em_ref[...]
        parts = [msg * e]
        if with_ones:
            parts.append(e)
            parts.append(jnp.zeros((BKE, P - 33), jnp.float32))
        out_ref[...] = jnp.concatenate(parts, axis=1)

    return pl.pallas_call(
        body,
        out_shape=jax.ShapeDtypeStruct((N_EDGES, P), jnp.float32),
        grid=(N_EDGES // BKE,),
        in_specs=[
            pl.BlockSpec((BKE, 32), lambda i: (i, 0)),
            pl.BlockSpec((BKE, 4), lambda i: (i, 0)),
            pl.BlockSpec((BKE, 1), lambda i: (i, 0)),
            pl.BlockSpec((1, 128), lambda i: (0, 0)),
            pl.BlockSpec((32, 256), lambda i: (0, 0)),
        ],
        out_specs=pl.BlockSpec((BKE, P), lambda i: (i, 0)),
    )(xs, rel, em, inv2, Wc)


def _tc_reduce(parts, b, n, bk, cnt_col=None, invc=None):
    P = parts.shape[2]
    first = cnt_col is not None

    def body(p_ref, *rest):
        if first:
            y_ref, st_ref, iv_ref = rest
            cnt = p_ref[0, :, cnt_col:cnt_col + 1] + p_ref[1, :, cnt_col:cnt_col + 1]
            iv = 1.0 / jnp.maximum(cnt, 1.0)
            iv_ref[...] = iv
        else:
            ic_ref, y_ref, st_ref = rest
            iv = ic_ref[...]
        agg = p_ref[0, :, :b] + p_ref[1, :, :b]
        a = agg * iv
        y = jnp.where(a > 0, a, jnp.exp(jnp.minimum(a, 0.0)) - 1.0)
        y_ref[...] = y

        @pl.when(pl.program_id(0) == 0)
        def _():
            st_ref[...] = jnp.zeros((8, 128), jnp.float32)

        st_ref[0:1, :] += _pad128(jnp.sum(y, axis=0, keepdims=True), b)
        st_ref[1:2, :] += _pad128(jnp.sum(y * y, axis=0, keepdims=True), b)

    outs = [jax.ShapeDtypeStruct((n, b), jnp.float32),
            jax.ShapeDtypeStruct((8, 128), jnp.float32)]
    out_specs = [pl.BlockSpec((bk, b), lambda i: (i, 0)),
                 pl.BlockSpec((8, 128), lambda i: (0, 0))]
    in_specs = [pl.BlockSpec((2, bk, P), lambda i: (0, i, 0))]
    args = [parts]
    if first:
        outs.append(jax.ShapeDtypeStruct((n, 1), jnp.float32))
        out_specs.append(pl.BlockSpec((bk, 1), lambda i: (i, 0)))
    else:
        in_specs.append(pl.BlockSpec((bk, 1), lambda i: (i, 0)))
        args.append(invc)
    return pl.pallas_call(
        body,
        out_shape=tuple(outs),
        grid=(n // bk,),
        in_specs=in_specs,
        out_specs=tuple(out_specs),
    )(*args)


def _tc_apply(y, scale, shift, n, b, bk, res=None):
    def body(y_ref, sc_ref, sh_ref, *rest):
        if res is None:
            (o_ref,) = rest
            o_ref[...] = y_ref[...] * sc_ref[...] + sh_ref[...]
        else:
            r_ref, o_ref = rest
            o_ref[...] = y_ref[...] * sc_ref[...] + sh_ref[...] + r_ref[...]

    in_specs = [pl.BlockSpec((bk, b), lambda i: (i, 0)),
                pl.BlockSpec((1, b), lambda i: (0, 0)),
                pl.BlockSpec((1, b), lambda i: (0, 0))]
    args = [y, scale, shift]
    if res is not None:
        in_specs.append(pl.BlockSpec((bk, b), lambda i: (i, 0)))
        args.append(res)
    return pl.pallas_call(
        body,
        out_shape=jax.ShapeDtypeStruct((n, b), jnp.float32),
        grid=(n // bk,),
        in_specs=in_specs,
        out_specs=pl.BlockSpec((bk, b), lambda i: (i, 0)),
    )(*args)


def _tc_cluster(pos, batch2):
    def body(pos_ref, b_ref, cl_ref, p16_ref, hist_ref):
        p = pos_ref[...]
        bt = b_ref[...]
        ix = jnp.clip(jnp.floor(p[:, 0:1] / VSX).astype(jnp.int32), 0, NX - 1)
        iy = jnp.clip(jnp.floor(p[:, 1:2] / VSY).astype(jnp.int32), 0, NY - 1)
        cl_ref[...] = bt * (NX * NY) + ix * NY + iy
        p16_ref[...] = jnp.concatenate(
            [p, jnp.ones((BKN, 1), jnp.float32),
             jnp.zeros((BKN, 12), jnp.float32)], axis=1)

        @pl.when(pl.program_id(0) == 0)
        def _():
            hist_ref[...] = jnp.zeros((8, 128), jnp.float32)

        oh = (bt == lax.broadcasted_iota(jnp.int32, (1, 16), 1)).astype(jnp.float32)
        hist_ref[0:1, :] += _pad128(jnp.sum(oh, axis=0, keepdims=True), 16)

    return pl.pallas_call(
        body,
        out_shape=(jax.ShapeDtypeStruct((N_NODES, 1), jnp.int32),
                   jax.ShapeDtypeStruct((N_NODES, 16), jnp.float32),
                   jax.ShapeDtypeStruct((8, 128), jnp.float32)),
        grid=(N_NODES // BKN,),
        in_specs=[pl.BlockSpec((BKN, 3), lambda i: (i, 0)),
                  pl.BlockSpec((BKN, 1), lambda i: (i, 0))],
        out_specs=(pl.BlockSpec((BKN, 1), lambda i: (i, 0)),
                   pl.BlockSpec((BKN, 16), lambda i: (i, 0)),
                   pl.BlockSpec((8, 128), lambda i: (0, 0))),
    )(pos, batch2)


def _tc_poolepi(pxp, posp):
    def body(px_ref, ps_ref, px_o, pp_o, cl2_o):
        m = jnp.maximum(px_ref[0], px_ref[1])
        px_o[...] = jnp.where(m > -1.0e37, m, 0.0)
        s = ps_ref[0] + ps_ref[1]
        cnt = jnp.maximum(s[:, 3:4], 1.0)
        pp = s[:, 0:3] / cnt
        pp_o[...] = jnp.concatenate([pp, jnp.zeros((BKC, 13), jnp.float32)], axis=1)
        jx = jnp.clip(jnp.floor(pp[:, 0:1] / 0.25).astype(jnp.int32), 0, 3)
        jy = jnp.clip(jnp.floor(pp[:, 1:2] / 0.25).astype(jnp.int32), 0, 3)
        rows = (pl.program_id(0) * BKC
                + lax.broadcasted_iota(jnp.int32, (BKC, 1), 0))
        cl2_o[...] = (rows // (NX * NY)) * 16 + jx * 4 + jy

    return pl.pallas_call(
        body,
        out_shape=(jax.ShapeDtypeStruct((C, 32), jnp.float32),
                   jax.ShapeDtypeStruct((C, 16), jnp.float32),
                   jax.ShapeDtypeStruct((C, 1), jnp.int32)),
        grid=(C // BKC,),
        in_specs=[pl.BlockSpec((2, BKC, 32), lambda i: (0, i, 0)),
                  pl.BlockSpec((2, BKC, 16), lambda i: (0, i, 0))],
        out_specs=(pl.BlockSpec((BKC, 32), lambda i: (i, 0)),
                   pl.BlockSpec((BKC, 16), lambda i: (i, 0)),
                   pl.BlockSpec((BKC, 1), lambda i: (i, 0))),
    )(pxp, posp)


def _tc_relmask(pps, ppd, psrc, pdst):
    def body(ps_ref, pd_ref, s_ref, d_ref, rel_o, em_o, mp_o):
        rel = pd_ref[:, 0:4] - ps_ref[:, 0:4]
        em = (s_ref[...] != d_ref[...]).astype(jnp.float32)
        rel_o[...] = rel
        em_o[...] = em

        @pl.when(pl.program_id(0) == 0)
        def _():
            mp_o[...] = jnp.zeros((8, 128), jnp.float32)

        mx = jnp.max(jnp.abs(rel * em), axis=0, keepdims=True)
        mp_o[0:1, :] = jnp.maximum(mp_o[0:1, :], _pad128(mx, 4))

    return pl.pallas_call(
        body,
        out_shape=(jax.ShapeDtypeStruct((N_EDGES, 4), jnp.float32),
                   jax.ShapeDtypeStruct((N_EDGES, 1), jnp.float32),
                   jax.ShapeDtypeStruct((8, 128), jnp.float32)),
        grid=(N_EDGES // BKE,),
        in_specs=[pl.BlockSpec((BKE, 16), lambda i: (i, 0)),
                  pl.BlockSpec((BKE, 16), lambda i: (i, 0)),
                  pl.BlockSpec((BKE, 1), lambda i: (i, 0)),
                  pl.BlockSpec((BKE, 1), lambda i: (i, 0))],
        out_specs=(pl.BlockSpec((BKE, 4), lambda i: (i, 0)),
                   pl.BlockSpec((BKE, 1), lambda i: (i, 0)),
                   pl.BlockSpec((8, 128), lambda i: (0, 0))),
    )(pps, ppd, psrc, pdst)


def _tc_fc(fxp, fc_w):
    def body(fx_ref, w_ref, o_ref):
        m = jnp.maximum(fx_ref[0], fx_ref[1])
        fx = jnp.where(m > -1.0e37, m, 0.0)
        o_ref[...] = jnp.dot(fx, w_ref[...], preferred_element_type=jnp.float32)

    return pl.pallas_call(
        body,
        out_shape=jax.ShapeDtypeStruct((16, 2), jnp.float32),
    )(fxp, fc_w)


def _bn_affine(stats, gamma, beta, n, b):
    s = stats[0, :b]
    ss = stats[1, :b]
    mean = s / n
    var = ss / n - mean * mean
    scale = gamma / jnp.sqrt(var + 1e-5)
    shift = beta - mean * scale
    return scale.reshape(1, b), shift.reshape(1, b)


def _tile_bounds(starts16, ends16):
    mids = (starts16 + ends16) // 2
    a = jnp.stack([(starts16 // 8) * 8, mids], 1)
    bb = jnp.stack([(mids // 8) * 8, ends16], 1)
    tb = jnp.zeros((NW, 16), jnp.int32)
    tb = tb.at[0::2, 0:2].set(a)
    tb = tb.at[1::2, 0:2].set(bb)
    return tb.reshape(NW * 16)


def _wcat(W):
    return jnp.transpose(W, (1, 0, 2)).reshape(W.shape[1], 8 * W.shape[2])


def kernel(x, pos, edge_attr, W1, W2, W3, W4, W5, W6, W7, gamma1, gamma2,
           gamma3, gamma4, gamma5, gamma6, gamma7, beta1, beta2, beta3, beta4,
           beta5, beta6, beta7, fc_w, edge_index, batch):
    src = edge_index[0]
    dst = edge_index[1]
    batch2 = batch.astype(jnp.int32).reshape(N_NODES, 1)

    cl, pos16, hist = _tc_cluster(pos, batch2)
    clf = cl.reshape(N_NODES)

    def fine_layer(table, a, b, Wc, gamma, beta, cnt_col=None, invc=None,
                   res=None):
        xs = _sc_gather(table, src, a, jnp.float32, use=True)
        msg = _tc_edge_fine(xs, edge_attr, Wc, a, b, cnt_col is not None)
        parts = _sc_scatter_add(msg, dst, N_NODES)
        out = _tc_reduce(parts, b, N_NODES, BKN, cnt_col=cnt_col, invc=invc)
        if cnt_col is not None:
            y, st, ic = out
        else:
            (y, st), ic = out, invc
        sc, sh = _bn_affine(st, gamma, beta, N_NODES, b)
        h = _tc_apply(y, sc, sh, N_NODES, b, BKN, res=res)
        return h, ic

    h1, invc = fine_layer(x, 1, 8, _wcat(W1), gamma1, beta1, cnt_col=8)
    h2, _ = fine_layer(h1, 8, 16, _wcat(W2), gamma2, beta2, invc=invc)
    h3, _ = fine_layer(h2, 16, 16, _wcat(W3), gamma3, beta3, invc=invc)
    h4r, _ = fine_layer(h3, 16, 16, _wcat(W4), gamma4, beta4, invc=invc,
                        res=h2)
    h5, _ = fine_layer(h4r, 16, 32, _wcat(W5), gamma5, beta5, invc=invc)

    MPAD = 50176
    pos16p = jnp.pad(pos16, ((0, MPAD - N_NODES), (0, 0)))
    clp = jnp.pad(clf, (0, MPAD - N_NODES))
    posparts = _sc_scatter_add(pos16p, clp, C)

    TPAD = N_NODES + 512
    h5p = jnp.pad(h5, ((0, TPAD - N_NODES), (0, 0)))
    clbig = jnp.pad(clf, (0, TPAD - N_NODES), constant_values=1 << 30)
    starts = jnp.concatenate([jnp.zeros((1,), jnp.int32),
                              jnp.cumsum(hist[0, 0:16]).astype(jnp.int32)])
    tb1 = _tile_bounds(starts[:16], starts[1:17])
    pxp = _sc_segmax(h5p, clbig, tb1, NX * NY)

    px, ppos16, cl2 = _tc_poolepi(pxp, posparts)

    psrc = _sc_gather(cl, src, 1, jnp.int32)
    pdst = _sc_gather(cl, dst, 1, jnp.int32)
    psf = psrc.reshape(N_EDGES)
    pdf = pdst.reshape(N_EDGES)
    pps = _sc_gather(ppos16, psf, 16, jnp.float32)
    ppd = _sc_gather(ppos16, pdf, 16, jnp.float32)
    rel, em, mpart = _tc_relmask(pps, ppd, psrc, pdst)
    mmax = jnp.maximum(jnp.max(mpart[0, 0:4]), 1e-9)
    inv2 = jnp.full((1, 128), 1.0 / (2.0 * mmax), jnp.float32)

    xs6 = _sc_gather(px, psf, 32, jnp.float32)
    m6 = _tc_edge_coarse(xs6, rel, em, inv2, _wcat(W6), True)
    p6parts = _sc_scatter_add(m6, pdf, C)
    y6, st6, invc6 = _tc_reduce(p6parts, 32, C, BKC, cnt_col=32)
    sc6, sh6 = _bn_affine(st6, gamma6, beta6, C, 32)
    p6 = _tc_apply(y6, sc6, sh6, C, 32, BKC)

    xs7 = _sc_gather(p6, psf, 32, jnp.float32)
    m7 = _tc_edge_coarse(xs7, rel, em, inv2, _wcat(W7), False)
    p7parts = _sc_scatter_add(m7, pdf, C)
    y7, st7 = _tc_reduce(p7parts, 32, C, BKC, invc=invc6)
    sc7, sh7 = _bn_affine(st7, gamma7, beta7, C, 32)
    p7r = _tc_apply(y7, sc7, sh7, C, 32, BKC, res=px)

    CPAD = C + 512
    p7p = jnp.pad(p7r, ((0, CPAD - C), (0, 0)))
    cl2big = jnp.pad(cl2.reshape(C), (0, CPAD - C), constant_values=1 << 30)
    cst = (jnp.arange(17, dtype=jnp.int32) * (NX * NY))
    tb2 = _tile_bounds(cst[:16], cst[1:17])
    fxp = _sc_segmax(p7p, cl2big, tb2, 16)

    out = _tc_fc(fxp.reshape(2, 16, 512), fc_w)
    return out


# speedup vs baseline: 2.0386x; 1.0373x over previous
import jax
import jax.numpy as jnp
from jax import lax
from jax.experimental import pallas as pl
from jax.experimental.pallas import tpu as pltpu
from jax.experimental.pallas import tpu_sc as plsc

N_NODES = 50000
N_EDGES = 800000
B = 16
NX = 22
NY = 22
VSX = 16.0 / 346.0
VSY = 12.0 / 260.0
C = B * NX * NY

NC, NS = 2, 16
NW = NC * NS
CH = 128
NEG = -3.0e38

_SC_PARAMS = pltpu.CompilerParams(use_tc_tiling_on_sc=False)


def _mesh():
    return plsc.VectorSubcoreMesh(
        core_axis_name="c", subcore_axis_name="s", num_cores=NC, num_subcores=NS)


def _sc_gather(table, idx, dtype):
    return _sc_gather_real(table, idx, table.shape[1], dtype)


def _sc_gather_real(table, idx, D, dtype):
    M = idx.shape[0]
    per = M // NW
    assert M % NW == 0 and per % 8 == 0
    nfull, tail = divmod(per, CH)
    assert tail % 8 == 0

    def body(table_h, idx_h, out_h, idxA, idxB, rowsA, rowsB, idxT, rowsT,
             semA, semB):
        cid = lax.axis_index("c")
        sid = lax.axis_index("s")
        base = (sid * NC + cid) * per

        def idx_load(buf, c):
            pltpu.sync_copy(idx_h.at[pl.ds(base + c * CH, CH)], buf)

        def g_start(ib, rb, sem):
            pltpu.async_copy(table_h.at[ib], rb, sem)

        def g_wait(ib, rb, sem):
            pltpu.make_async_copy(table_h.at[ib], rb, sem).wait()

        def store(rb, c):
            pltpu.sync_copy(rb, out_h.at[pl.ds(base + c * CH, CH), :])

        if nfull > 0:
            idx_load(idxA, 0)
            g_start(idxA, rowsA, semA)

            @pl.loop(0, (nfull + 1) // 2)
            def _(t):
                c1 = 2 * t + 1

                @pl.when(c1 < nfull)
                def _():
                    idx_load(idxB, c1)
                    g_start(idxB, rowsB, semB)

                g_wait(idxA, rowsA, semA)
                store(rowsA, 2 * t)

                @pl.when(c1 < nfull)
                def _():
                    @pl.when(c1 + 1 < nfull)
                    def _():
                        idx_load(idxA, c1 + 1)
                        g_start(idxA, rowsA, semA)

                    g_wait(idxB, rowsB, semB)
                    store(rowsB, c1)

        if tail:
            pltpu.sync_copy(idx_h.at[pl.ds(base + nfull * CH, tail)], idxT)
            pltpu.async_copy(table_h.at[idxT], rowsT, semA).wait()
            pltpu.sync_copy(rowsT, out_h.at[pl.ds(base + nfull * CH, tail), :])

    fn = pl.kernel(
        body,
        out_type=jax.ShapeDtypeStruct((M, D), dtype),
        mesh=_mesh(),
        compiler_params=_SC_PARAMS,
        scratch_types=[
            pltpu.VMEM((CH,), jnp.int32),
            pltpu.VMEM((CH,), jnp.int32),
            pltpu.VMEM((CH, D), dtype),
            pltpu.VMEM((CH, D), dtype),
            pltpu.VMEM((tail or 8,), jnp.int32),
            pltpu.VMEM((tail or 8, D), dtype),
            pltpu.SemaphoreType.DMA,
            pltpu.SemaphoreType.DMA,
        ],
    )
    return fn(table, idx)


def _sc_scatter_add(msg, dst, nrows):
    M, P = msg.shape
    assert P % 16 == 0 and nrows % NS == 0
    per = M // NW
    assert M % NW == 0 and per % 8 == 0
    nfull, tail = divmod(per, CH)
    assert tail % 8 == 0
    rows_t = nrows // NS
    znf, ztail = divmod(rows_t, CH)

    def body(msg_h, dst_h, out_h, acc, idxA, idxB, rowsA, rowsB, idxT, rowsT,
             zb, semA, semB):
        cid = lax.axis_index("c")
        sid = lax.axis_index("s")
        base = cid * (M // NC) + sid * per
        r0 = sid * rows_t

        for r in range(CH):
            for col in range(0, P, 16):
                zb[r, pl.ds(col, 16)] = jnp.zeros((16,), jnp.float32)

        @pl.loop(0, znf)
        def _(j):
            pltpu.sync_copy(zb, acc.at[pl.ds(r0 + j * CH, CH), :])

        if ztail:
            pltpu.sync_copy(zb.at[pl.ds(0, ztail), :],
                            acc.at[pl.ds(r0 + znf * CH, ztail), :])

        plsc.subcore_barrier()

        def idx_load(buf, c):
            pltpu.sync_copy(dst_h.at[pl.ds(base + c * CH, CH)], buf)

        def m_start(rb, c, sem):
            pltpu.async_copy(msg_h.at[pl.ds(base + c * CH, CH), :], rb, sem)

        def m_wait(rb, c, sem):
            pltpu.make_async_copy(
                msg_h.at[pl.ds(base + c * CH, CH), :], rb, sem).wait()

        def scat(ib, rb):
            pltpu.sync_copy(rb, acc.at[ib], add=True)

        if nfull > 0:
            idx_load(idxA, 0)
            m_start(rowsA, 0, semA)

            @pl.loop(0, (nfull + 1) // 2)
            def _(t):
                c0 = 2 * t
                c1 = c0 + 1

                @pl.when(c1 < nfull)
                def _():
                    idx_load(idxB, c1)
                    m_start(rowsB, c1, semB)

                m_wait(rowsA, c0, semA)
                scat(idxA, rowsA)

                @pl.when(c1 < nfull)
                def _():
                    @pl.when(c1 + 1 < nfull)
                    def _():
                        idx_load(idxA, c1 + 1)
                        m_start(rowsA, c1 + 1, semA)

                    m_wait(rowsB, c1, semB)
                    scat(idxB, rowsB)

        if tail:
            pltpu.sync_copy(dst_h.at[pl.ds(base + nfull * CH, tail)], idxT)
            pltpu.async_copy(
                msg_h.at[pl.ds(base + nfull * CH, tail), :], rowsT, semA).wait()
            pltpu.sync_copy(rowsT, acc.at[idxT], add=True)

        plsc.subcore_barrier()

        @pl.loop(0, znf)
        def _(j):
            pltpu.sync_copy(acc.at[pl.ds(r0 + j * CH, CH), :],
                            out_h.at[cid, pl.ds(r0 + j * CH, CH), :])

        if ztail:
            pltpu.sync_copy(acc.at[pl.ds(r0 + znf * CH, ztail), :],
                            out_h.at[cid, pl.ds(r0 + znf * CH, ztail), :])

    fn = pl.kernel(
        body,
        out_type=jax.ShapeDtypeStruct((NC, nrows, P), jnp.float32),
        mesh=_mesh(),
        compiler_params=_SC_PARAMS,
        scratch_types=[
            pltpu.VMEM_SHARED((nrows, P), jnp.float32),
            pltpu.VMEM((CH,), jnp.int32),
            pltpu.VMEM((CH,), jnp.int32),
            pltpu.VMEM((CH, P), jnp.float32),
            pltpu.VMEM((CH, P), jnp.float32),
            pltpu.VMEM((tail or 8,), jnp.int32),
            pltpu.VMEM((tail or 8, P), jnp.float32),
            pltpu.VMEM((CH, P), jnp.float32),
            pltpu.SemaphoreType.DMA,
            pltpu.SemaphoreType.DMA,
        ],
    )
    return fn(msg, dst)


def _sc_segmax(hpad, clpad, tb, cps):
    CHN = 512

    def body(h_h, cl_h, tb_h, out_h, acc, hbuf, clbuf, tbv, sem):
        cid = lax.axis_index("c")
        sid = lax.axis_index("s")
        wid = sid * NC + cid
        s = wid // 2
        half = wid % 2
        lo = s * cps

        pltpu.sync_copy(tb_h.at[pl.ds(pl.multiple_of(wid * 16, 8), 16)], tbv)
        bvec = tbv[pl.ds(0, 16)]
        tstart = bvec[0]
        tend = bvec[1]

        @pl.loop(0, cps)
        def _(j):
            acc[j, pl.ds(0, 16)] = jnp.full((16,), NEG, jnp.float32)
            acc[j, pl.ds(16, 16)] = jnp.full((16,), NEG, jnp.float32)

        nch = (tend - tstart + CHN - 1) // CHN

        @pl.loop(0, nch)
        def _(c):
            r0 = pl.multiple_of(tstart + c * CHN, 8)
            pltpu.async_copy(h_h.at[pl.ds(r0, CHN), :], hbuf, sem)
            pltpu.sync_copy(cl_h.at[pl.ds(r0, CHN)], clbuf)
            pltpu.make_async_copy(h_h.at[pl.ds(r0, CHN), :], hbuf, sem).wait()

            @pl.loop(0, CHN // 16)
            def _(v):
                clvec = clbuf[pl.ds(v * 16, 16)]
                for lane in range(16):
                    clv = clvec[lane]
                    i = v * 16 + lane

                    @pl.when((clv >= lo) & (clv < lo + cps))
                    def _():
                        j = clv - lo
                        acc[j, pl.ds(0, 16)] = jnp.maximum(
                            acc[j, pl.ds(0, 16)], hbuf[i, pl.ds(0, 16)])
                        acc[j, pl.ds(16, 16)] = jnp.maximum(
                            acc[j, pl.ds(16, 16)], hbuf[i, pl.ds(16, 16)])

        pltpu.sync_copy(acc, out_h.at[half, pl.ds(s * cps, cps), :])

    fn = pl.kernel(
        body,
        out_type=jax.ShapeDtypeStruct((2, B * cps, 32), jnp.float32),
        mesh=_mesh(),
        compiler_params=_SC_PARAMS,
        scratch_types=[
            pltpu.VMEM((cps, 32), jnp.float32),
            pltpu.VMEM((CHN, 32), jnp.float32),
            pltpu.VMEM((CHN,), jnp.int32),
            pltpu.VMEM((16,), jnp.int32),
            pltpu.SemaphoreType.DMA,
        ],
    )
    return fn(hpad, clpad, tb)


BKE = 3200
BKN = 2000
BKC = 968


def _pad128(row, width):
    return jnp.concatenate(
        [row, jnp.zeros((1, 128 - width), jnp.float32)], axis=1)


def _tc_w8(ea):
    def body(ea_ref, out_ref):
        u = jnp.clip(ea_ref[...], 0.0, 1.0)
        u0, u1, u2 = u[:, 0:1], u[:, 1:2], u[:, 2:3]
        cols = []
        for k in range(8):
            w = (u0 if k & 1 else 1.0 - u0)
            w = w * (u1 if k & 2 else 1.0 - u1)
            w = w * (u2 if k & 4 else 1.0 - u2)
            cols.append(w)
        cols.append(jnp.zeros((BKE, 8), jnp.float32))
        out_ref[...] = jnp.concatenate(cols, axis=1)

    return pl.pallas_call(
        body,
        out_shape=jax.ShapeDtypeStruct((N_EDGES, 16), jnp.float32),
        grid=(N_EDGES // BKE,),
        in_specs=[pl.BlockSpec((BKE, 3), lambda i: (i, 0))],
        out_specs=pl.BlockSpec((BKE, 16), lambda i: (i, 0)),
    )(ea)


def _wmsg(t, w8, b, n):
    out = jnp.zeros((n, b), jnp.float32)
    for k in range(8):
        out = out + w8[:, k:k + 1] * t[:, k * b:(k + 1) * b]
    return out


def _tc_edge_fine(xs, w8, Wc, a, b, with_ones):
    P = 16 if b <= 8 or with_ones else b
    ap = xs.shape[1]

    def body(xs_ref, w8_ref, wc_ref, out_ref):
        xv = xs_ref[:, :a]
        if a == 1:
            t = xv * wc_ref[0:1, :]
        else:
            t = jnp.dot(xv, wc_ref[...], preferred_element_type=jnp.float32)
        msg = _wmsg(t, w8_ref[...], b, BKE)
        parts = [msg]
        if with_ones:
            parts.append(jnp.ones((BKE, 1), jnp.float32))
        pad = P - sum(p.shape[1] for p in parts)
        if pad:
            parts.append(jnp.zeros((BKE, pad), jnp.float32))
        out_ref[...] = jnp.concatenate(parts, axis=1)

    return pl.pallas_call(
        body,
        out_shape=jax.ShapeDtypeStruct((N_EDGES, P), jnp.float32),
        grid=(N_EDGES // BKE,),
        in_specs=[pl.BlockSpec((BKE, ap), lambda i: (i, 0)),
                  pl.BlockSpec((BKE, 16), lambda i: (i, 0)),
                  pl.BlockSpec((a, 8 * b), lambda i: (0, 0))],
        out_specs=pl.BlockSpec((BKE, P), lambda i: (i, 0)),
    )(xs, w8, Wc)


def _tc_edge_coarse(xs, rel, inv2, Wc, with_ones):
    b = 32
    P = 48 if with_ones else 32

    def body(xs_ref, rel_ref, inv_ref, wc_ref, out_ref):
        iv = inv_ref[0, 0]
        rel = rel_ref[...]
        u = jnp.clip(rel * iv + 0.5, 0.0, 1.0)
        u0, u1, u2 = u[:, 0:1], u[:, 1:2], u[:, 2:3]
        e = rel[:, 3:4]
        t = jnp.dot(xs_ref[...], wc_ref[...], preferred_element_type=jnp.float32)
        w8 = []
        for k in range(8):
            w = (u0 if k & 1 else 1.0 - u0)
            w = w * (u1 if k & 2 else 1.0 - u1)
            w = w * (u2 if k & 4 else 1.0 - u2)
            w8.append(w)
        msg = _wmsg(t, jnp.concatenate(w8, axis=1), b, BKE)
        parts = [msg * e]
        if with_ones:
            parts.append(e)
            parts.append(jnp.zeros((BKE, P - 33), jnp.float32))
        out_ref[...] = jnp.concatenate(parts, axis=1)

    return pl.pallas_call(
        body,
        out_shape=jax.ShapeDtypeStruct((N_EDGES, P), jnp.float32),
        grid=(N_EDGES // BKE,),
        in_specs=[
            pl.BlockSpec((BKE, 32), lambda i: (i, 0)),
            pl.BlockSpec((BKE, 16), lambda i: (i, 0)),
            pl.BlockSpec((1, 128), lambda i: (0, 0)),
            pl.BlockSpec((32, 256), lambda i: (0, 0)),
        ],
        out_specs=pl.BlockSpec((BKE, P), lambda i: (i, 0)),
    )(xs, rel, inv2, Wc)


def _tc_reduce(parts, b, n, bk, cnt_col=None, invc=None):
    P = parts.shape[2]
    first = cnt_col is not None

    def body(p_ref, *rest):
        if first:
            y_ref, st_ref, iv_ref = rest
            cnt = p_ref[0, :, cnt_col:cnt_col + 1] + p_ref[1, :, cnt_col:cnt_col + 1]
            iv = 1.0 / jnp.maximum(cnt, 1.0)
            iv_ref[...] = iv
        else:
            ic_ref, y_ref, st_ref = rest
            iv = ic_ref[...]
        agg = p_ref[0, :, :b] + p_ref[1, :, :b]
        a = agg * iv
        y = jnp.where(a > 0, a, jnp.exp(jnp.minimum(a, 0.0)) - 1.0)
        y_ref[...] = y

        @pl.when(pl.program_id(0) == 0)
        def _():
            st_ref[...] = jnp.zeros((8, 128), jnp.float32)

        st_ref[0:1, :] += _pad128(jnp.sum(y, axis=0, keepdims=True), b)
        st_ref[1:2, :] += _pad128(jnp.sum(y * y, axis=0, keepdims=True), b)

    outs = [jax.ShapeDtypeStruct((n, b), jnp.float32),
            jax.ShapeDtypeStruct((8, 128), jnp.float32)]
    out_specs = [pl.BlockSpec((bk, b), lambda i: (i, 0)),
                 pl.BlockSpec((8, 128), lambda i: (0, 0))]
    in_specs = [pl.BlockSpec((2, bk, P), lambda i: (0, i, 0))]
    args = [parts]
    if first:
        outs.append(jax.ShapeDtypeStruct((n, 1), jnp.float32))
        out_specs.append(pl.BlockSpec((bk, 1), lambda i: (i, 0)))
    else:
        in_specs.append(pl.BlockSpec((bk, 1), lambda i: (i, 0)))
        args.append(invc)
    return pl.pallas_call(
        body,
        out_shape=tuple(outs),
        grid=(n // bk,),
        in_specs=in_specs,
        out_specs=tuple(out_specs),
    )(*args)


def _tc_apply(y, scale, shift, n, b, bk, res=None, bp=None):
    bp = bp or b

    def body(y_ref, sc_ref, sh_ref, *rest):
        if res is None:
            (o_ref,) = rest
        else:
            r_ref, o_ref = rest
        h = y_ref[...] * sc_ref[...] + sh_ref[...]
        if res is not None:
            h = h + r_ref[:, :b]
        if bp != b:
            h = jnp.concatenate([h, jnp.zeros((bk, bp - b), jnp.float32)], 1)
        o_ref[...] = h

    in_specs = [pl.BlockSpec((bk, b), lambda i: (i, 0)),
                pl.BlockSpec((1, b), lambda i: (0, 0)),
                pl.BlockSpec((1, b), lambda i: (0, 0))]
    args = [y, scale, shift]
    if res is not None:
        in_specs.append(pl.BlockSpec((bk, res.shape[1]), lambda i: (i, 0)))
        args.append(res)
    return pl.pallas_call(
        body,
        out_shape=jax.ShapeDtypeStruct((n, bp), jnp.float32),
        grid=(n // bk,),
        in_specs=in_specs,
        out_specs=pl.BlockSpec((bk, bp), lambda i: (i, 0)),
    )(*args)


def _tc_cluster(pos, batch2):
    def body(pos_ref, b_ref, cl_ref, p16_ref, hist_ref):
        p = pos_ref[...]
        bt = b_ref[...]
        ix = jnp.clip(jnp.floor(p[:, 0:1] / VSX).astype(jnp.int32), 0, NX - 1)
        iy = jnp.clip(jnp.floor(p[:, 1:2] / VSY).astype(jnp.int32), 0, NY - 1)
        cl = bt * (NX * NY) + ix * NY + iy
        cl_ref[...] = jnp.concatenate(
            [cl, jnp.zeros((BKN, 15), jnp.int32)], axis=1)
        p16_ref[...] = jnp.concatenate(
            [p, jnp.ones((BKN, 1), jnp.float32),
             jnp.zeros((BKN, 12), jnp.float32)], axis=1)

        @pl.when(pl.program_id(0) == 0)
        def _():
            hist_ref[...] = jnp.zeros((8, 128), jnp.float32)

        oh = (bt == lax.broadcasted_iota(jnp.int32, (1, 16), 1)).astype(jnp.float32)
        hist_ref[0:1, :] += _pad128(jnp.sum(oh, axis=0, keepdims=True), 16)

    return pl.pallas_call(
        body,
        out_shape=(jax.ShapeDtypeStruct((N_NODES, 16), jnp.int32),
                   jax.ShapeDtypeStruct((N_NODES, 16), jnp.float32),
                   jax.ShapeDtypeStruct((8, 128), jnp.float32)),
        grid=(N_NODES // BKN,),
        in_specs=[pl.BlockSpec((BKN, 3), lambda i: (i, 0)),
                  pl.BlockSpec((BKN, 1), lambda i: (i, 0))],
        out_specs=(pl.BlockSpec((BKN, 16), lambda i: (i, 0)),
                   pl.BlockSpec((BKN, 16), lambda i: (i, 0)),
                   pl.BlockSpec((8, 128), lambda i: (0, 0))),
    )(pos, batch2)


def _tc_poolepi(pxp, posp):
    def body(px_ref, ps_ref, px_o, pp_o, cl2_o):
        m = jnp.maximum(px_ref[0], px_ref[1])
        px_o[...] = jnp.where(m > -1.0e37, m, 0.0)
        s = ps_ref[0] + ps_ref[1]
        cnt = jnp.maximum(s[:, 3:4], 1.0)
        pp = s[:, 0:3] / cnt
        pp_o[...] = jnp.concatenate([pp, jnp.zeros((BKC, 13), jnp.float32)], axis=1)
        jx = jnp.clip(jnp.floor(pp[:, 0:1] / 0.25).astype(jnp.int32), 0, 3)
        jy = jnp.clip(jnp.floor(pp[:, 1:2] / 0.25).astype(jnp.int32), 0, 3)
        rows = (pl.program_id(0) * BKC
                + lax.broadcasted_iota(jnp.int32, (BKC, 1), 0))
        cl2_o[...] = (rows // (NX * NY)) * 16 + jx * 4 + jy

    return pl.pallas_call(
        body,
        out_shape=(jax.ShapeDtypeStruct((C, 32), jnp.float32),
                   jax.ShapeDtypeStruct((C, 16), jnp.float32),
                   jax.ShapeDtypeStruct((C, 1), jnp.int32)),
        grid=(C // BKC,),
        in_specs=[pl.BlockSpec((2, BKC, 32), lambda i: (0, i, 0)),
                  pl.BlockSpec((2, BKC, 16), lambda i: (0, i, 0))],
        out_specs=(pl.BlockSpec((BKC, 32), lambda i: (i, 0)),
                   pl.BlockSpec((BKC, 16), lambda i: (i, 0)),
                   pl.BlockSpec((BKC, 1), lambda i: (i, 0))),
    )(pxp, posp)


def _tc_relmask(pps, ppd, psrc, pdst):
    def body(ps_ref, pd_ref, s_ref, d_ref, rel_o, mp_o):
        rel3 = pd_ref[:, 0:3] - ps_ref[:, 0:3]
        em = (s_ref[:, 0:1] != d_ref[:, 0:1]).astype(jnp.float32)
        rel_o[...] = jnp.concatenate(
            [rel3, em, jnp.zeros((BKE, 12), jnp.float32)], axis=1)

        @pl.when(pl.program_id(0) == 0)
        def _():
            mp_o[...] = jnp.zeros((8, 128), jnp.float32)

        mx = jnp.max(jnp.abs(rel3 * em), axis=0, keepdims=True)
        mp_o[0:1, :] = jnp.maximum(mp_o[0:1, :], _pad128(mx, 3))

    return pl.pallas_call(
        body,
        out_shape=(jax.ShapeDtypeStruct((N_EDGES, 16), jnp.float32),
                   jax.ShapeDtypeStruct((8, 128), jnp.float32)),
        grid=(N_EDGES // BKE,),
        in_specs=[pl.BlockSpec((BKE, 16), lambda i: (i, 0)),
                  pl.BlockSpec((BKE, 16), lambda i: (i, 0)),
                  pl.BlockSpec((BKE, 16), lambda i: (i, 0)),
                  pl.BlockSpec((BKE, 16), lambda i: (i, 0))],
        out_specs=(pl.BlockSpec((BKE, 16), lambda i: (i, 0)),
                   pl.BlockSpec((8, 128), lambda i: (0, 0))),
    )(pps, ppd, psrc, pdst)


def _tc_fc(fxp, fc_w):
    def body(fx_ref, w_ref, o_ref):
        m = jnp.maximum(fx_ref[0], fx_ref[1])
        fx = jnp.where(m > -1.0e37, m, 0.0)
        o_ref[...] = jnp.dot(fx, w_ref[...], preferred_element_type=jnp.float32)

    return pl.pallas_call(
        body,
        out_shape=jax.ShapeDtypeStruct((16, 2), jnp.float32),
    )(fxp, fc_w)


def _bn_affine(stats, gamma, beta, n, b):
    s = stats[0, :b]
    ss = stats[1, :b]
    mean = s / n
    var = ss / n - mean * mean
    scale = gamma / jnp.sqrt(var + 1e-5)
    shift = beta - mean * scale
    return scale.reshape(1, b), shift.reshape(1, b)


def _tile_bounds(starts16, ends16):
    mids = (starts16 + ends16) // 2
    a = jnp.stack([(starts16 // 8) * 8, mids], 1)
    bb = jnp.stack([(mids // 8) * 8, ends16], 1)
    tb = jnp.zeros((NW, 16), jnp.int32)
    tb = tb.at[0::2, 0:2].set(a)
    tb = tb.at[1::2, 0:2].set(bb)
    return tb.reshape(NW * 16)


def _wcat(W):
    return jnp.transpose(W, (1, 0, 2)).reshape(W.shape[1], 8 * W.shape[2])


def kernel(x, pos, edge_attr, W1, W2, W3, W4, W5, W6, W7, gamma1, gamma2,
           gamma3, gamma4, gamma5, gamma6, gamma7, beta1, beta2, beta3, beta4,
           beta5, beta6, beta7, fc_w, edge_index, batch):
    src = edge_index[0]
    dst = edge_index[1]
    batch2 = batch.astype(jnp.int32).reshape(N_NODES, 1)

    cl16, pos16, hist = _tc_cluster(pos, batch2)
    clf = cl16[:, 0]
    w8 = _tc_w8(edge_attr)
    x16 = jnp.pad(x, ((0, 0), (0, 15)))

    def fine_layer(table, a, b, Wc, gamma, beta, cnt_col=None, invc=None,
                   res=None, bp=None):
        xs = _sc_gather(table, src, jnp.float32)
        msg = _tc_edge_fine(xs, w8, Wc, a, b, cnt_col is not None)
        parts = _sc_scatter_add(msg, dst, N_NODES)
        out = _tc_reduce(parts, b, N_NODES, BKN, cnt_col=cnt_col, invc=invc)
        if cnt_col is not None:
            y, st, ic = out
        else:
            (y, st), ic = out, invc
        sc, sh = _bn_affine(st, gamma, beta, N_NODES, b)
        h = _tc_apply(y, sc, sh, N_NODES, b, BKN, res=res, bp=bp)
        return h, ic

    h1, invc = fine_layer(x16, 1, 8, _wcat(W1), gamma1, beta1, cnt_col=8,
                          bp=16)
    h2, _ = fine_layer(h1, 8, 16, _wcat(W2), gamma2, beta2, invc=invc)
    h3, _ = fine_layer(h2, 16, 16, _wcat(W3), gamma3, beta3, invc=invc)
    h4r, _ = fine_layer(h3, 16, 16, _wcat(W4), gamma4, beta4, invc=invc,
                        res=h2)
    h5, _ = fine_layer(h4r, 16, 32, _wcat(W5), gamma5, beta5, invc=invc)

    MPAD = 50176
    pos16p = jnp.pad(pos16, ((0, MPAD - N_NODES), (0, 0)))
    clp = jnp.pad(clf, (0, MPAD - N_NODES))
    posparts = _sc_scatter_add(pos16p, clp, C)

    TPAD = N_NODES + 512
    h5p = jnp.pad(h5, ((0, TPAD - N_NODES), (0, 0)))
    clbig = jnp.pad(clf, (0, TPAD - N_NODES), constant_values=1 << 30)
    starts = jnp.concatenate([jnp.zeros((1,), jnp.int32),
                              jnp.cumsum(hist[0, 0:16]).astype(jnp.int32)])
    tb1 = _tile_bounds(starts[:16], starts[1:17])
    pxp = _sc_segmax(h5p, clbig, tb1, NX * NY)

    px, ppos16, cl2 = _tc_poolepi(pxp, posparts)

    psrc = _sc_gather(cl16, src, jnp.int32)
    pdst = _sc_gather(cl16, dst, jnp.int32)
    psf = psrc[:, 0]
    pdf = pdst[:, 0]
    pps = _sc_gather(ppos16, psf, jnp.float32)
    ppd = _sc_gather(ppos16, pdf, jnp.float32)
    rel, mpart = _tc_relmask(pps, ppd, psrc, pdst)
    mmax = jnp.maximum(jnp.max(mpart[0, 0:4]), 1e-9)
    inv2 = jnp.full((1, 128), 1.0 / (2.0 * mmax), jnp.float32)

    xs6 = _sc_gather(px, psf, jnp.float32)
    m6 = _tc_edge_coarse(xs6, rel, inv2, _wcat(W6), True)
    p6parts = _sc_scatter_add(m6, pdf, C)
    y6, st6, invc6 = _tc_reduce(p6parts, 32, C, BKC, cnt_col=32)
    sc6, sh6 = _bn_affine(st6, gamma6, beta6, C, 32)
    p6 = _tc_apply(y6, sc6, sh6, C, 32, BKC)

    xs7 = _sc_gather(p6, psf, jnp.float32)
    m7 = _tc_edge_coarse(xs7, rel, inv2, _wcat(W7), False)
    p7parts = _sc_scatter_add(m7, pdf, C)
    y7, st7 = _tc_reduce(p7parts, 32, C, BKC, invc=invc6)
    sc7, sh7 = _bn_affine(st7, gamma7, beta7, C, 32)
    p7r = _tc_apply(y7, sc7, sh7, C, 32, BKC, res=px)

    CPAD = C + 512
    p7p = jnp.pad(p7r, ((0, CPAD - C), (0, 0)))
    cl2big = jnp.pad(cl2.reshape(C), (0, CPAD - C), constant_values=1 << 30)
    cst = (jnp.arange(17, dtype=jnp.int32) * (NX * NY))
    tb2 = _tile_bounds(cst[:16], cst[1:17])
    fxp = _sc_segmax(p7p, cl2big, tb2, 16)

    out = _tc_fc(fxp.reshape(2, 16, 512), fc_w)
    return out
